# Initial kernel scaffold; baseline (speedup 1.0000x reference)
import jax
import jax.numpy as jnp
from jax.experimental import pallas as pl


def kernel(x):
    raise NotImplementedError("write your pallas kernel here")


# speedup vs baseline: 1.0328x; 1.0328x over previous
import jax
import jax.numpy as jnp
from jax.experimental import pallas as pl

_NUM_STICK = 117
_NUM_TRIGGER = 99
_NUM_BUTTONS = 128
_WIDTH = 4 * _NUM_STICK + _NUM_TRIGGER + _NUM_BUTTONS
_OFFSETS = (0, _NUM_STICK, 2 * _NUM_STICK, 3 * _NUM_STICK,
            4 * _NUM_STICK, 4 * _NUM_STICK + _NUM_TRIGGER)

_ROWS = 2048


def _onehot_body(x_ref, o_ref):
    codes = x_ref[...]
    col = jax.lax.broadcasted_iota(jnp.int32, (_ROWS, _WIDTH), 1)
    acc = col == (codes[:, 0:1] + _OFFSETS[0])
    for c in range(1, 6):
        acc = jnp.logical_or(acc, col == (codes[:, c:c + 1] + _OFFSETS[c]))
    o_ref[...] = acc.astype(jnp.float32)


def kernel(x):
    b, t, ncomp = x.shape
    rows = b * t
    xf = x.reshape(rows, ncomp).astype(jnp.int32)
    grid = (rows // _ROWS,)
    out = pl.pallas_call(
        _onehot_body,
        grid=grid,
        in_specs=[pl.BlockSpec((_ROWS, ncomp), lambda i: (i, 0))],
        out_specs=pl.BlockSpec((_ROWS, _WIDTH), lambda i: (i, 0)),
        out_shape=jax.ShapeDtypeStruct((rows, _WIDTH), jnp.float32),
    )(xf)
    return out.reshape(b, t, _WIDTH)


# speedup vs baseline: 1.2507x; 1.2110x over previous
import jax
import jax.numpy as jnp
from jax.experimental import pallas as pl

_NUM_STICK = 117
_NUM_TRIGGER = 99
_NUM_BUTTONS = 128
_WIDTH = 4 * _NUM_STICK + _NUM_TRIGGER + _NUM_BUTTONS
_OFFSETS = (0, _NUM_STICK, 2 * _NUM_STICK, 3 * _NUM_STICK,
            4 * _NUM_STICK, 4 * _NUM_STICK + _NUM_TRIGGER)

_ROWS = 2048


_TILE_SEGS = ((0, 1), (1, 2), (2, 3), (3, 4), (4, 5), (5,))


def _onehot_body(x_ref, o_ref):
    codes = x_ref[...]
    for k, segs in enumerate(_TILE_SEGS):
        lo = 128 * k
        hi = min(lo + 128, _WIDTH)
        w = hi - lo
        col = jax.lax.broadcasted_iota(jnp.int32, (_ROWS, w), 1) + lo
        acc = col == (codes[:, segs[0]:segs[0] + 1] + _OFFSETS[segs[0]])
        for s in segs[1:]:
            acc = jnp.logical_or(acc, col == (codes[:, s:s + 1] + _OFFSETS[s]))
        o_ref[:, lo:hi] = acc.astype(jnp.float32)


def kernel(x):
    b, t, ncomp = x.shape
    rows = b * t
    xf = x.reshape(rows, ncomp).astype(jnp.int32)
    grid = (rows // _ROWS,)
    out = pl.pallas_call(
        _onehot_body,
        grid=grid,
        in_specs=[pl.BlockSpec((_ROWS, ncomp), lambda i: (i, 0))],
        out_specs=pl.BlockSpec((_ROWS, _WIDTH), lambda i: (i, 0)),
        out_shape=jax.ShapeDtypeStruct((rows, _WIDTH), jnp.float32),
    )(xf)
    return out.reshape(b, t, _WIDTH)


# speedup vs baseline: 1.9446x; 1.5549x over previous
import jax
import jax.numpy as jnp
from jax.experimental import pallas as pl

_NUM_STICK = 117
_NUM_TRIGGER = 99
_NUM_BUTTONS = 128
_WIDTH = 4 * _NUM_STICK + _NUM_TRIGGER + _NUM_BUTTONS
_OFFSETS = (0, _NUM_STICK, 2 * _NUM_STICK, 3 * _NUM_STICK,
            4 * _NUM_STICK, 4 * _NUM_STICK + _NUM_TRIGGER)

_TILE_SEGS = ((0, 1), (1, 2), (2, 3), (3, 4), (4, 5), (5,))

_BB = 128


def _onehot_body(x_ref, o_ref):
    codes = x_ref[...]
    t = codes.shape[1]
    for k, segs in enumerate(_TILE_SEGS):
        lo = 128 * k
        hi = min(lo + 128, _WIDTH)
        w = hi - lo
        col = jax.lax.broadcasted_iota(jnp.int32, (_BB, t, w), 2) + lo
        acc = col == (codes[:, :, segs[0]:segs[0] + 1] + _OFFSETS[segs[0]])
        for s in segs[1:]:
            acc = jnp.logical_or(
                acc, col == (codes[:, :, s:s + 1] + _OFFSETS[s]))
        o_ref[:, :, lo:hi] = acc.astype(jnp.float32)


def kernel(x):
    b, t, ncomp = x.shape
    grid = (b // _BB,)
    return pl.pallas_call(
        _onehot_body,
        grid=grid,
        in_specs=[pl.BlockSpec((_BB, t, ncomp), lambda i: (i, 0, 0))],
        out_specs=pl.BlockSpec((_BB, t, _WIDTH), lambda i: (i, 0, 0)),
        out_shape=jax.ShapeDtypeStruct((b, t, _WIDTH), jnp.float32),
    )(x.astype(jnp.int32))


# speedup vs baseline: 1.9472x; 1.0013x over previous
import jax
import jax.numpy as jnp
from jax.experimental import pallas as pl

_NUM_STICK = 117
_NUM_TRIGGER = 99
_NUM_BUTTONS = 128
_WIDTH = 4 * _NUM_STICK + _NUM_TRIGGER + _NUM_BUTTONS
_OFFSETS = (0, _NUM_STICK, 2 * _NUM_STICK, 3 * _NUM_STICK,
            4 * _NUM_STICK, 4 * _NUM_STICK + _NUM_TRIGGER)

_TILE_SEGS = ((0, 1), (1, 2), (2, 3), (3, 4), (4, 5), (5,))

_BB = 256


def _onehot_body(x_ref, o_ref):
    codes = x_ref[...]
    t = codes.shape[1]
    for k, segs in enumerate(_TILE_SEGS):
        lo = 128 * k
        hi = min(lo + 128, _WIDTH)
        w = hi - lo
        col = jax.lax.broadcasted_iota(jnp.int32, (_BB, t, w), 2) + lo
        acc = col == (codes[:, :, segs[0]:segs[0] + 1] + _OFFSETS[segs[0]])
        for s in segs[1:]:
            acc = jnp.logical_or(
                acc, col == (codes[:, :, s:s + 1] + _OFFSETS[s]))
        o_ref[:, :, lo:hi] = acc.astype(jnp.float32)


def kernel(x):
    b, t, ncomp = x.shape
    grid = (b // _BB,)
    return pl.pallas_call(
        _onehot_body,
        grid=grid,
        in_specs=[pl.BlockSpec((_BB, t, ncomp), lambda i: (i, 0, 0))],
        out_specs=pl.BlockSpec((_BB, t, _WIDTH), lambda i: (i, 0, 0)),
        out_shape=jax.ShapeDtypeStruct((b, t, _WIDTH), jnp.float32),
    )(x.astype(jnp.int32))
